# Initial kernel scaffold; baseline (speedup 1.0000x reference)
import jax
import jax.numpy as jnp
from jax.experimental import pallas as pl


def kernel(x, W0, W1, cheb_b, fc_w, fc_b, final_w, final_b, edge_index):
    raise NotImplementedError("write your pallas kernel here")


# speedup vs baseline: 4.6416x; 4.6416x over previous
import jax
import jax.numpy as jnp
from jax.experimental import pallas as pl
from jax.experimental.pallas import tpu as pltpu

_V = 16
_T = 200
_E = 32
_GCN = 32
_HID = 128
_VT = _V * _T


def _fold_kernel(ei_vec_ref, ei_smem_ref, W0_ref, W1_ref, cheb_ref, fcw_ref,
                 fcb_ref, G_ref, b1_ref, M_ref):
    src = ei_vec_ref[0:1, :]
    dst = ei_vec_ref[1:2, :]
    nodes = jax.lax.broadcasted_iota(jnp.int32, (_V, _E), 0)
    a_src = (src == nodes).astype(jnp.float32)
    a_dst = (dst == nodes).astype(jnp.float32)
    deg = jnp.sum(a_src, axis=1, keepdims=True)
    dinv = jnp.where(deg > 0, jax.lax.rsqrt(deg), 0.0)
    dinv_src = jnp.sum(a_src * dinv, axis=0, keepdims=True)
    dinv_dst = jnp.sum(a_dst * dinv, axis=0, keepdims=True)
    norm = -(dinv_src * dinv_dst)

    M_ref[...] = jnp.zeros((_V * _GCN, _HID), jnp.float32)
    for e in range(_E):
        s = ei_smem_ref[0, e]
        d = ei_smem_ref[1, e]
        M_ref[pl.ds(s * _GCN, _GCN), :] = (
            M_ref[pl.ds(s * _GCN, _GCN), :]
            + norm[0:1, e:e + 1] * fcw_ref[pl.ds(d * _GCN, _GCN), :])

    W0 = W0_ref[...]
    W1 = W1_ref[...]
    fsum = jnp.zeros((_GCN, _HID), jnp.float32)
    for u in range(_V):
        Fu = fcw_ref[_GCN * u:_GCN * (u + 1), :]
        fsum = fsum + Fu
        G_ref[_T * u:_T * (u + 1), :] = (
            jnp.dot(W0, Fu, preferred_element_type=jnp.float32)
            + jnp.dot(W1, M_ref[_GCN * u:_GCN * (u + 1), :],
                      preferred_element_type=jnp.float32))

    cheb = cheb_ref[...].reshape(1, _GCN)
    b1_ref[...] = (jnp.dot(cheb, fsum, preferred_element_type=jnp.float32)
                   + fcb_ref[...].reshape(1, _HID))


def _mlp_kernel(x_ref, G_ref, b1_ref, fw_ref, fb_ref, out_ref):
    lat = jnp.dot(x_ref[...], G_ref[...],
                  preferred_element_type=jnp.float32) + b1_ref[...]
    a = jnp.maximum(lat, 0.0)
    o = jnp.dot(a, fw_ref[...], preferred_element_type=jnp.float32)
    out_ref[...] = jax.nn.sigmoid(o + fb_ref[...])


def kernel(x, W0, W1, cheb_b, fc_w, fc_b, final_w, final_b, edge_index):
    B = x.shape[0]
    xf = x.reshape(B, _VT)
    fb = final_b.reshape(1, 1)

    G, b1 = pl.pallas_call(
        _fold_kernel,
        out_shape=[
            jax.ShapeDtypeStruct((_VT, _HID), jnp.float32),
            jax.ShapeDtypeStruct((1, _HID), jnp.float32),
        ],
        in_specs=[
            pl.BlockSpec(memory_space=pltpu.VMEM),
            pl.BlockSpec(memory_space=pltpu.SMEM),
            pl.BlockSpec(memory_space=pltpu.VMEM),
            pl.BlockSpec(memory_space=pltpu.VMEM),
            pl.BlockSpec(memory_space=pltpu.VMEM),
            pl.BlockSpec(memory_space=pltpu.VMEM),
            pl.BlockSpec(memory_space=pltpu.VMEM),
        ],
        scratch_shapes=[pltpu.VMEM((_V * _GCN, _HID), jnp.float32)],
    )(edge_index, edge_index, W0, W1, cheb_b, fc_w, fc_b)

    BLK = 512
    out = pl.pallas_call(
        _mlp_kernel,
        grid=(B // BLK,),
        in_specs=[
            pl.BlockSpec((BLK, _VT), lambda i: (i, 0)),
            pl.BlockSpec((_VT, _HID), lambda i: (0, 0)),
            pl.BlockSpec((1, _HID), lambda i: (0, 0)),
            pl.BlockSpec((_HID, 1), lambda i: (0, 0)),
            pl.BlockSpec((1, 1), lambda i: (0, 0)),
        ],
        out_specs=pl.BlockSpec((BLK, 1), lambda i: (i, 0)),
        out_shape=jax.ShapeDtypeStruct((B, 1), jnp.float32),
    )(xf, G, b1, final_w, fb)
    return out
